# Initial kernel scaffold; baseline (speedup 1.0000x reference)
import jax
import jax.numpy as jnp
from jax.experimental import pallas as pl


def kernel(nodes, neighbours, table, W1, b1, W2, b2, W3, b3):
    raise NotImplementedError("write your pallas kernel here")


# speedup vs baseline: 3.4759x; 3.4759x over previous
import functools

import jax
import jax.numpy as jnp
from jax import lax
from jax.experimental import pallas as pl
from jax.experimental.pallas import tpu as pltpu
from jax.experimental.pallas import tpu_sc as plsc

B = 4096
DEG = 32
D = 256
GATHER_WINDOW = 128

def _sc_gather(table, idx2d):
    n_rows = idx2d.shape[1]
    assert n_rows % GATHER_WINDOW == 0

    @functools.partial(
        pl.kernel,
        out_type=jax.ShapeDtypeStruct((n_rows, table.shape[1]), table.dtype),
        mesh=plsc.VectorSubcoreMesh(core_axis_name="c", subcore_axis_name="s"),
    )
    def k(table_hbm, idx_hbm, out_hbm):
        def body(i_vmem, o_vmem):
            pltpu.sync_copy(table_hbm.at[i_vmem.at[0]], o_vmem)

        pltpu.emit_pipeline(
            body,
            grid=(n_rows // GATHER_WINDOW,),
            in_specs=[pl.BlockSpec((1, GATHER_WINDOW), index_map=lambda i: (0, i))],
            out_specs=[
                pl.BlockSpec((GATHER_WINDOW, table.shape[1]), index_map=lambda i: (i, 0))
            ],
            core_axis_name=("c", "s"),
            dimension_semantics=(pltpu.PARALLEL,),
        )(idx_hbm, out_hbm)

    return k(table, idx2d)


def _mlp_body(nref, uref, w1_ref, w2_ref, b1_ref, b2_ref, w3_ref, oref):
    bb = uref.shape[0]
    n = nref[...]
    u = uref[...]
    w1 = w1_ref[...]
    w1n = w1[:, :D]
    w1u = w1[:, D:]
    cdims = (((1,), (1,)), ((), ()))
    n1 = lax.dot_general(n, w1n, cdims, preferred_element_type=jnp.float32)
    u1 = lax.dot_general(u, w1u, cdims, preferred_element_type=jnp.float32)
    u1 = u1 + b1_ref[...]
    h1 = jnp.maximum(n1.reshape(bb, DEG, D) + u1[:, None, :], 0.0)
    h2 = lax.dot_general(
        h1.reshape(bb * DEG, D), w2_ref[...], cdims,
        preferred_element_type=jnp.float32,
    )
    h2 = jnp.maximum(h2 + b2_ref[...], 0.0)
    s = jnp.sum(h2.reshape(bb, DEG, D) * w3_ref[...][None], axis=2)
    s = s - jnp.max(s, axis=1, keepdims=True)
    e = jnp.exp(s)
    att = e / jnp.sum(e, axis=1, keepdims=True)
    out = jnp.sum(n.reshape(bb, DEG, D) * att[:, :, None], axis=1)
    oref[...] = out


def _tc_mlp(gathered, W1, W2, b1, b2, W3, block_b):
    nblocks = B // block_b
    full = lambda shape: pl.BlockSpec(shape, lambda i: tuple(0 for _ in shape))
    return pl.pallas_call(
        _mlp_body,
        grid=(nblocks,),
        in_specs=[
            pl.BlockSpec((block_b * DEG, D), lambda i: (i + B // (block_b * DEG), 0)),
            pl.BlockSpec((block_b, D), lambda i: (i, 0)),
            full((D, 2 * D)),
            full((D, D)),
            full((1, D)),
            full((1, D)),
            full((1, D)),
        ],
        out_specs=pl.BlockSpec((block_b, D), lambda i: (i, 0)),
        out_shape=jax.ShapeDtypeStruct((B, D), jnp.float32),
    )(gathered, gathered, W1, W2, b1, b2, W3)


def kernel(nodes, neighbours, table, W1, b1, W2, b2, W3, b3):
    del b3
    idx = jnp.concatenate(
        [nodes.astype(jnp.int32), neighbours.reshape(-1).astype(jnp.int32)]
    ).reshape(1, -1)
    gathered = _sc_gather(table, idx)
    block_b = 128
    return _tc_mlp(
        gathered,
        W1,
        W2,
        b1.reshape(1, D),
        b2.reshape(1, D),
        W3,
        block_b,
    )
